# baseline (speedup 1.0000x reference)
import functools

import jax
import jax.numpy as jnp
from jax import lax
from jax.experimental import pallas as pl
from jax.experimental.pallas import tpu as pltpu
from jax.experimental.pallas import tpu_sc as plsc

NUM_EXPERTS = 64
TOP_K = 8
BT = 512
T_TOK = 8192

_INT_MIN = -(2 ** 31)
_INT_MAX = 2 ** 31 - 1


def _mm_body(x_ref, w_ref, logits_ref):
    logits_ref[...] = jax.lax.dot_general(
        w_ref[...], x_ref[...], (((1,), (1,)), ((), ())),
        preferred_element_type=jnp.float32,
    )


def _tc_logits_t(hidden_states, W_gate):
    T, H = hidden_states.shape
    E = W_gate.shape[0]
    nb = T // BT
    return pl.pallas_call(
        _mm_body,
        grid=(nb,),
        in_specs=[
            pl.BlockSpec((BT, H), lambda i: (i, 0)),
            pl.BlockSpec((E, H), lambda i: (0, 0)),
        ],
        out_specs=pl.BlockSpec((E, BT), lambda i: (0, i)),
        out_shape=jax.ShapeDtypeStruct((E, T), jnp.float32),
    )(hidden_states, W_gate)


def _msc(v):
    return jnp.where(v >= 0, v, jnp.int32(_INT_MIN) - v)


def _sc_topk(logits_t):
    info = plsc.get_sparse_core_info()
    nc, ns = info.num_cores, info.num_subcores
    nw = nc * ns
    tok_w = T_TOK // nw

    mesh = plsc.VectorSubcoreMesh(core_axis_name="c", subcore_axis_name="s")

    @functools.partial(
        pl.kernel,
        mesh=mesh,
        out_type=[
            jax.ShapeDtypeStruct((TOP_K, T_TOK), jnp.int32),
            jax.ShapeDtypeStruct((TOP_K, T_TOK), jnp.float32),
        ],
        scratch_types=[
            pltpu.VMEM((NUM_EXPERTS, tok_w), jnp.float32),
            pltpu.VMEM((NUM_EXPERTS, tok_w), jnp.int32),
            pltpu.VMEM((TOP_K, tok_w), jnp.int32),
            pltpu.VMEM((TOP_K, tok_w), jnp.float32),
        ],
    )
    def k(lt_hbm, idx_hbm, wts_hbm, lg_v, kv_v, oi_v, ow_v):
        wid = lax.axis_index("s") * nc + lax.axis_index("c")
        base = wid * tok_w
        pltpu.sync_copy(lt_hbm.at[:, pl.ds(base, tok_w)], lg_v)

        def body(g, _):
            ds = pl.ds(g * 16, 16)
            for e in range(NUM_EXPERTS):
                a = lg_v[e, ds]
                b = jax.lax.bitcast_convert_type(a, jnp.int32)
                kv_v[e, ds] = (_msc(b) & jnp.int32(~63)) | jnp.int32(
                    NUM_EXPERTS - 1 - e
                )

            sel_keys = []
            last = jnp.full((16,), _INT_MAX, jnp.int32)
            for _k in range(TOP_K):
                m = jnp.full((16,), _INT_MIN, jnp.int32)
                for e in range(NUM_EXPERTS):
                    c = kv_v[e, ds]
                    m = jnp.maximum(m, jnp.where(c < last, c, jnp.int32(_INT_MIN)))
                sel_keys.append(m)
                last = m

            ixs = [jnp.int32(NUM_EXPERTS - 1) - (kk & jnp.int32(63)) for kk in sel_keys]

            neg_inf = jnp.float32(-jnp.inf)
            evs = [jnp.full((16,), neg_inf, jnp.float32) for _ in range(TOP_K)]
            for e in range(NUM_EXPERTS):
                ck = kv_v[e, ds]
                cv = lg_v[e, ds]
                for _k in range(TOP_K):
                    evs[_k] = jnp.where(ck == sel_keys[_k], cv, evs[_k])

            for parity in (0, 1):
                for p in range(parity, TOP_K - 1, 2):
                    a_v, b_v = evs[p], evs[p + 1]
                    a_i, b_i = ixs[p], ixs[p + 1]
                    beat = (b_v > a_v) | ((b_v == a_v) & (b_i < a_i))
                    evs[p] = jnp.where(beat, b_v, a_v)
                    evs[p + 1] = jnp.where(beat, a_v, b_v)
                    ixs[p] = jnp.where(beat, b_i, a_i)
                    ixs[p + 1] = jnp.where(beat, a_i, b_i)

            es = [jnp.exp(v - evs[0]) for v in evs]
            s = es[0]
            for _k in range(1, TOP_K):
                s = s + es[_k]
            for _k in range(TOP_K):
                oi_v[_k, ds] = ixs[_k]
                ow_v[_k, ds] = es[_k] / s
            return _

        jax.lax.fori_loop(0, tok_w // 16, body, None)

        pltpu.sync_copy(oi_v, idx_hbm.at[:, pl.ds(base, tok_w)])
        pltpu.sync_copy(ow_v, wts_hbm.at[:, pl.ds(base, tok_w)])

    return k(logits_t)


def kernel(hidden_states, W_gate):
    if hidden_states.ndim == 3:
        hidden_states = hidden_states.reshape(-1, hidden_states.shape[-1])
    logits_t = _tc_logits_t(hidden_states, W_gate)
    idx_t, wts_t = _sc_topk(logits_t)
    return (logits_t.T, idx_t.T, wts_t.T)


# speedup vs baseline: 2.5216x; 2.5216x over previous
import jax
import jax.numpy as jnp
from jax.experimental import pallas as pl
from jax.experimental.pallas import tpu as pltpu

NUM_EXPERTS = 64
TOP_K = 8
BT = 512


def _m(v):
    return jnp.where(v >= 0, v, jnp.int32(-(2**31)) - v)


def _rot_rows(a, s):
    return jnp.concatenate([a[s:], a[:s]], axis=0)


def _max8(a):
    a = jnp.maximum(a[:32], a[32:])
    a = jnp.maximum(a[:16], a[16:])
    a = jnp.maximum(a[:8], a[8:])
    a = jnp.maximum(a, _rot_rows(a, 4))
    a = jnp.maximum(a, _rot_rows(a, 2))
    a = jnp.maximum(a, _rot_rows(a, 1))
    return a


def _topk8_t(logits_t, idx_ref, wts_ref):
    row = jax.lax.broadcasted_iota(jnp.int32, (NUM_EXPERTS, BT), 0)
    row8 = jax.lax.broadcasted_iota(jnp.int32, (TOP_K, BT), 0)
    b = jax.lax.bitcast_convert_type(logits_t, jnp.int32)
    key = (_m(b) & jnp.int32(~63)) | (jnp.int32(NUM_EXPERTS - 1) - row)
    cur = jax.lax.bitcast_convert_type(_m(key), jnp.float32)

    neg_inf = jnp.float32(-jnp.inf)
    kt = jnp.zeros((TOP_K, BT), jnp.float32)
    evt = jnp.zeros((TOP_K, BT), jnp.float32)
    for k in range(TOP_K):
        mk8 = _max8(cur)
        mk = jnp.concatenate([mk8] * (NUM_EXPERTS // TOP_K), axis=0)
        onehot = cur == mk
        ev8 = _max8(jnp.where(onehot, logits_t, neg_inf))
        sel = row8 == k
        kt = jnp.where(sel, mk8, kt)
        evt = jnp.where(sel, ev8, evt)
        cur = jnp.where(onehot, neg_inf, cur)

    kk = _m(jax.lax.bitcast_convert_type(kt, jnp.int32))
    ixt = jnp.int32(NUM_EXPERTS - 1) - (kk & jnp.int32(63))

    for parity in (0, 1):
        ev_r = _rot_rows(evt, 1)
        ix_r = _rot_rows(ixt, 1)
        beat = (ev_r > evt) | ((ev_r == evt) & (ix_r < ixt))
        can = (row8 % 2 == parity) & (row8 < TOP_K - 1)
        swap = jnp.where(beat & can, jnp.int32(1), jnp.int32(0))
        swap_l = _rot_rows(swap, TOP_K - 1)
        ev_l = _rot_rows(evt, TOP_K - 1)
        ix_l = _rot_rows(ixt, TOP_K - 1)
        swap_l = jnp.where(row8 == 0, 0, swap_l)
        evt = jnp.where(swap == 1, ev_r, jnp.where(swap_l == 1, ev_l, evt))
        ixt = jnp.where(swap == 1, ix_r, jnp.where(swap_l == 1, ix_l, ixt))

    idx_ref[...] = ixt
    mx = jnp.concatenate([evt[0:1]] * TOP_K, axis=0)
    e = jnp.exp(evt - mx)
    s = e
    s = s + _rot_rows(s, 4)
    s = s + _rot_rows(s, 2)
    s = s + _rot_rows(s, 1)
    wts_ref[...] = e / s


def _router_body(x_ref, w_ref, logits_ref, idx_ref, wts_ref, prev_ref):
    _topk8_t(prev_ref[...], idx_ref, wts_ref)
    logits_t = jax.lax.dot_general(
        w_ref[...], x_ref[...], (((1,), (1,)), ((), ())),
        preferred_element_type=jnp.float32,
    )
    logits_ref[...] = logits_t
    prev_ref[...] = logits_t


def kernel(hidden_states, W_gate):
    if hidden_states.ndim == 3:
        hidden_states = hidden_states.reshape(-1, hidden_states.shape[-1])
    T, H = hidden_states.shape
    E = W_gate.shape[0]
    nb = T // BT
    last = nb - 1
    logits_t, idx_t, wts_t = pl.pallas_call(
        _router_body,
        grid=(nb + 1,),
        in_specs=[
            pl.BlockSpec((BT, H), lambda i: (jnp.minimum(i, last), 0)),
            pl.BlockSpec((E, H), lambda i: (0, 0)),
        ],
        out_specs=[
            pl.BlockSpec((E, BT), lambda i: (0, jnp.minimum(i, last))),
            pl.BlockSpec((TOP_K, BT), lambda i: (0, jnp.maximum(i - 1, 0))),
            pl.BlockSpec((TOP_K, BT), lambda i: (0, jnp.maximum(i - 1, 0))),
        ],
        out_shape=[
            jax.ShapeDtypeStruct((E, T), jnp.float32),
            jax.ShapeDtypeStruct((TOP_K, T), jnp.int32),
            jax.ShapeDtypeStruct((TOP_K, T), jnp.float32),
        ],
        scratch_shapes=[pltpu.VMEM((E, BT), jnp.float32)],
    )(hidden_states, W_gate)
    return (logits_t.T, idx_t.T, wts_t.T)
